# baseline (speedup 1.0000x reference)
import functools

import jax
import jax.numpy as jnp
from jax import lax
from jax.experimental import pallas as pl
from jax.experimental.pallas import tpu as pltpu
from jax.experimental.pallas import tpu_sc as plsc

_N = 50000
_E = 800000
_D = 64
_H = 32
_NC = 2
_NS = 16
_NW = _NC * _NS
_CH = 1000
_CPW = _E // (_NW * _CH)
_SUB = 8
_SUBL = _CH // _SUB
_BLK_N = 2000
_BLK_E = 2000
_P = 8

_f32 = jnp.float32


def _silu(v):
    return v * (1.0 / (1.0 + jnp.exp(-v)))


def _embed_body(x_ref, be_ref, wx_ref, wd_ref, ws_ref, h_ref, a_ref, b_ref):
    h = jnp.dot(x_ref[...], wx_ref[...], preferred_element_type=_f32)
    h = h + be_ref[...]
    h_ref[...] = h
    a_ref[...] = jnp.dot(h, wd_ref[...], preferred_element_type=_f32)
    b_ref[...] = jnp.dot(h, ws_ref[...], preferred_element_type=_f32)


def _edge_body(ga_ref, gb_ref, pd_ref, ps_ref, ea_ref,
               wr_ref, we_ref, b1_ref, w2_ref, b2_ref, wc_ref, bc_ref,
               ma_ref, mb_ref, gd_ref):
    diff = pd_ref[...] - ps_ref[...]
    r2 = jnp.sum(diff * diff, axis=1, keepdims=True)
    pre = (ga_ref[...] + gb_ref[...] + r2 * wr_ref[...]
           + jnp.dot(ea_ref[...], we_ref[...], preferred_element_type=_f32)
           + b1_ref[...])
    m = _silu(pre)
    m = _silu(jnp.dot(m, w2_ref[...], preferred_element_type=_f32) + b2_ref[...])
    gamma4 = jnp.dot(m, wc_ref[...], preferred_element_type=_f32) + bc_ref[...]
    gamma = gamma4[:, 0:1]
    ma_ref[...] = m[:, :_H // 2]
    mb_ref[...] = m[:, _H // 2:]
    col = lax.broadcasted_iota(jnp.int32, (1, _P), 1)
    one3 = (col == 3).astype(_f32)
    gd_ref[...] = gamma * diff + one3


def _edge_call(ga, gb, pd, ps, ea, wr, we, b1, w2, b2, wc4, bc4):
    grid = _E // _BLK_E
    full = lambda r, c: pl.BlockSpec((r, c), lambda i: (0, 0))
    row32 = pl.BlockSpec((_BLK_E, _H), lambda i: (i, 0))
    rowp = pl.BlockSpec((_BLK_E, _P), lambda i: (i, 0))
    row4 = pl.BlockSpec((_BLK_E, 4), lambda i: (i, 0))
    row16 = pl.BlockSpec((_BLK_E, _H // 2), lambda i: (i, 0))
    return pl.pallas_call(
        _edge_body,
        grid=(grid,),
        in_specs=[row32, row32, rowp, rowp, row4,
                  full(1, _H), full(4, _H), full(1, _H),
                  full(_H, _H), full(1, _H), full(_H, 4), full(1, 4)],
        out_specs=[row16, row16, rowp],
        out_shape=[
            jax.ShapeDtypeStruct((_E, _H // 2), _f32),
            jax.ShapeDtypeStruct((_E, _H // 2), _f32),
            jax.ShapeDtypeStruct((_E, _P), _f32),
        ],
    )(ga, gb, pd, ps, ea, wr, we, b1, w2, b2, wc4, bc4)


def _node_body(h_ref, ms0_ref, ms1_ref, ca_ref, pp_ref,
               w1h_ref, w1m0_ref, w1m1_ref, b1_ref, w2_ref, b2_ref,
               wdn_ref, wsn_ref,
               h_out, pp_out, a_out, b_out):
    cacc = ca_ref[...]
    deg = jnp.maximum(cacc[:, 3:4], 1.0)
    inv = 1.0 / deg
    hn = _silu(jnp.dot(h_ref[...], w1h_ref[...], preferred_element_type=_f32)
               + jnp.dot(ms0_ref[...] * inv, w1m0_ref[...], preferred_element_type=_f32)
               + jnp.dot(ms1_ref[...] * inv, w1m1_ref[...], preferred_element_type=_f32)
               + b1_ref[...])
    hnew = jnp.dot(hn, w2_ref[...], preferred_element_type=_f32) + b2_ref[...]
    h_out[...] = hnew
    col = lax.broadcasted_iota(jnp.int32, (1, _P), 1)
    mask3 = (col < 3).astype(_f32)
    pp_out[...] = pp_ref[...] + (cacc * mask3) * inv
    a_out[...] = jnp.dot(hnew, wdn_ref[...], preferred_element_type=_f32)
    b_out[...] = jnp.dot(hnew, wsn_ref[...], preferred_element_type=_f32)


def _node_call(h, ms0, ms1, ca, pp, w1h, w1m0, w1m1, b1, w2, b2, wdn, wsn):
    grid = _N // _BLK_N
    full = lambda r, c: pl.BlockSpec((r, c), lambda i: (0, 0))
    row32 = pl.BlockSpec((_BLK_N, _H), lambda i: (i, 0))
    row16 = pl.BlockSpec((_BLK_N, _H // 2), lambda i: (i, 0))
    rowp = pl.BlockSpec((_BLK_N, _P), lambda i: (i, 0))
    return pl.pallas_call(
        _node_body,
        grid=(grid,),
        in_specs=[row32, row16, row16, rowp, rowp,
                  full(_H, _H), full(_H // 2, _H), full(_H // 2, _H),
                  full(1, _H),
                  full(_H, _H), full(1, _H), full(_H, _H), full(_H, _H)],
        out_specs=[row32, rowp, row32, row32],
        out_shape=[
            jax.ShapeDtypeStruct((_N, _H), _f32),
            jax.ShapeDtypeStruct((_N, _P), _f32),
            jax.ShapeDtypeStruct((_N, _H), _f32),
            jax.ShapeDtypeStruct((_N, _H), _f32),
        ],
    )(h, ms0, ms1, ca, pp, w1h, w1m0, w1m1, b1, w2, b2, wdn, wsn)


def _node_final_body(h_ref, ms0_ref, ms1_ref, ca_ref, pp_ref,
                     w1h_ref, w1m0_ref, w1m1_ref, b1_ref, w2_ref, b2_ref,
                     ecw_ref, ecb_ref, efw_ref, efb_ref,
                     ec_out, ef_out, pp_out):
    cacc = ca_ref[...]
    deg = jnp.maximum(cacc[:, 3:4], 1.0)
    inv = 1.0 / deg
    hn = _silu(jnp.dot(h_ref[...], w1h_ref[...], preferred_element_type=_f32)
               + jnp.dot(ms0_ref[...] * inv, w1m0_ref[...], preferred_element_type=_f32)
               + jnp.dot(ms1_ref[...] * inv, w1m1_ref[...], preferred_element_type=_f32)
               + b1_ref[...])
    hnew = jnp.dot(hn, w2_ref[...], preferred_element_type=_f32) + b2_ref[...]
    col = lax.broadcasted_iota(jnp.int32, (1, _P), 1)
    mask3 = (col < 3).astype(_f32)
    pp_out[...] = pp_ref[...] + (cacc * mask3) * inv
    ec_out[...] = jnp.dot(hnew, ecw_ref[...], preferred_element_type=_f32) + ecb_ref[...]
    ef_out[...] = jnp.dot(hnew, efw_ref[...], preferred_element_type=_f32) + efb_ref[...]


def _node_final_call(h, ms0, ms1, ca, pp, w1h, w1m0, w1m1, b1, w2, b2,
                     ecw4, ecb4, efw, efb):
    grid = _N // _BLK_N
    full = lambda r, c: pl.BlockSpec((r, c), lambda i: (0, 0))
    row32 = pl.BlockSpec((_BLK_N, _H), lambda i: (i, 0))
    row16 = pl.BlockSpec((_BLK_N, _H // 2), lambda i: (i, 0))
    rowp = pl.BlockSpec((_BLK_N, _P), lambda i: (i, 0))
    row4 = pl.BlockSpec((_BLK_N, 4), lambda i: (i, 0))
    row64 = pl.BlockSpec((_BLK_N, _D), lambda i: (i, 0))
    return pl.pallas_call(
        _node_final_body,
        grid=(grid,),
        in_specs=[row32, row16, row16, rowp, rowp,
                  full(_H, _H), full(_H // 2, _H), full(_H // 2, _H),
                  full(1, _H),
                  full(_H, _H), full(1, _H),
                  full(_H, 4), full(1, 4), full(_H, _D), full(1, _D)],
        out_specs=[row4, row64, rowp],
        out_shape=[
            jax.ShapeDtypeStruct((_N, 4), _f32),
            jax.ShapeDtypeStruct((_N, _D), _f32),
            jax.ShapeDtypeStruct((_N, _P), _f32),
        ],
    )(h, ms0, ms1, ca, pp, w1h, w1m0, w1m1, b1, w2, b2, ecw4, ecb4, efw, efb)


_MESH = plsc.VectorSubcoreMesh(core_axis_name="c", subcore_axis_name="s")


@functools.partial(
    pl.kernel,
    out_type=[
        jax.ShapeDtypeStruct((_E, _H), _f32),
        jax.ShapeDtypeStruct((_E, _H), _f32),
        jax.ShapeDtypeStruct((_E, _P), _f32),
        jax.ShapeDtypeStruct((_E, _P), _f32),
    ],
    mesh=_MESH,
    scratch_types=[
        pltpu.VMEM((_SUB, _SUBL), jnp.int32),
        pltpu.VMEM((_SUB, _SUBL), jnp.int32),
        pltpu.VMEM((_CH, _H), _f32),
        pltpu.VMEM((_CH, _H), _f32),
        pltpu.VMEM((_CH, _P), _f32),
        pltpu.VMEM((_CH, _P), _f32),
        pltpu.SemaphoreType.DMA,
    ],
    compiler_params=pltpu.CompilerParams(use_tc_tiling_on_sc=False),
)
def _sc_gather(a_hbm, b_hbm, pp_hbm, dst3, src3,
               ga_out, gb_out, pd_out, ps_out,
               idxd, idxs, bufa, bufb, bufpd, bufps, sem):
    c = lax.axis_index("c")
    s = lax.axis_index("s")
    wid = s * _NC + c

    def chunk(j, carry):
        cg = wid * _CPW + j
        base = cg * _CH
        pltpu.sync_copy(dst3.at[cg], idxd)
        pltpu.sync_copy(src3.at[cg], idxs)

        def sub(jj, carry2):
            rows = pl.ds(jj * _SUBL, _SUBL)
            d1 = pltpu.async_copy(a_hbm.at[idxd.at[jj]], bufa.at[rows], sem)
            d2 = pltpu.async_copy(b_hbm.at[idxs.at[jj]], bufb.at[rows], sem)
            d3 = pltpu.async_copy(pp_hbm.at[idxd.at[jj]], bufpd.at[rows], sem)
            d4 = pltpu.async_copy(pp_hbm.at[idxs.at[jj]], bufps.at[rows], sem)
            d1.wait()
            d2.wait()
            d3.wait()
            d4.wait()
            return carry2

        lax.fori_loop(0, _SUB, sub, 0)
        pltpu.sync_copy(bufa, ga_out.at[pl.ds(base, _CH)])
        pltpu.sync_copy(bufb, gb_out.at[pl.ds(base, _CH)])
        pltpu.sync_copy(bufpd, pd_out.at[pl.ds(base, _CH)])
        pltpu.sync_copy(bufps, ps_out.at[pl.ds(base, _CH)])
        return carry

    lax.fori_loop(0, _CPW, chunk, 0)


_NCHUNK = _E // _CH
_CPS = _NCHUNK // _NS


@functools.partial(
    pl.kernel,
    out_type=[
        jax.ShapeDtypeStruct((_NC, _N, _H // 2), _f32),
        jax.ShapeDtypeStruct((_N, _P), _f32),
    ],
    mesh=_MESH,
    scratch_types=[
        pltpu.VMEM((_SUB, _SUBL), jnp.int32),
        pltpu.VMEM((_CH, _H // 2), _f32),
        pltpu.VMEM((_CH, _P), _f32),
        pltpu.VMEM_SHARED((_N, _H // 2), _f32),
        pltpu.VMEM_SHARED((_N, _P), _f32),
        pltpu.SemaphoreType.DMA,
    ],
    compiler_params=pltpu.CompilerParams(use_tc_tiling_on_sc=False),
)
def _sc_scatter(ma_hbm, mb_hbm, gd_hbm, dst3, z16, zp, ms_out, ca_out,
                idxd, bufm, bufg, msacc, caacc, sem):
    c = lax.axis_index("c")
    s = lax.axis_index("s")

    @pl.when(s == 0)
    def _init():
        pltpu.sync_copy(z16, msacc)

    @pl.when((s == 1) & (c == 1))
    def _init2():
        pltpu.sync_copy(zp, caacc)

    plsc.subcore_barrier()

    def chunk(j, carry):
        cg = s * _CPS + j
        rows = pl.ds(cg * _CH, _CH)
        pltpu.sync_copy(dst3.at[cg], idxd)

        @pl.when(c == 0)
        def _stage_a():
            pltpu.sync_copy(ma_hbm.at[rows], bufm)

        @pl.when(c == 1)
        def _stage_b():
            pltpu.sync_copy(mb_hbm.at[rows], bufm)
            pltpu.sync_copy(gd_hbm.at[rows], bufg)

        def sub_m(jj, carry2):
            sub = pl.ds(jj * _SUBL, _SUBL)
            pltpu.sync_copy(bufm.at[sub], msacc.at[idxd.at[jj]], add=True)
            return carry2

        lax.fori_loop(0, _SUB, sub_m, 0)

        @pl.when(c == 1)
        def _scat_g():
            def sub_g(jj, carry2):
                sub = pl.ds(jj * _SUBL, _SUBL)
                pltpu.sync_copy(bufg.at[sub], caacc.at[idxd.at[jj]], add=True)
                return carry2

            lax.fori_loop(0, _SUB, sub_g, 0)

        return carry

    lax.fori_loop(0, _CPS, chunk, 0)
    plsc.subcore_barrier()

    @pl.when(s == 0)
    def _writeout():
        pltpu.sync_copy(msacc, ms_out.at[c])

    @pl.when((s == 1) & (c == 1))
    def _writeout2():
        pltpu.sync_copy(caacc, ca_out)


def kernel(x, pos, edge_index, edge_attr, t, params):
    src = edge_index[0].astype(jnp.int32)
    dst = edge_index[1].astype(jnp.int32)
    dst3 = dst.reshape(_NW * _CPW, _SUB, _SUBL)
    src3 = src.reshape(_NW * _CPW, _SUB, _SUBL)
    pospad = jnp.pad(pos.astype(_f32), ((0, 0), (0, _P - 3)))
    ea = edge_attr.astype(_f32)

    z16 = jnp.zeros((_N, _H // 2), _f32)
    zp = jnp.zeros((_N, _P), _f32)

    lp0 = params["layers"][0]
    bias_eff = (t[0] * params["in_W"][_D] + params["in_b"]).reshape(1, _H)

    grid = _N // _BLK_N
    full = lambda r, c: pl.BlockSpec((r, c), lambda i: (0, 0))
    h, A, B = pl.pallas_call(
        _embed_body,
        grid=(grid,),
        in_specs=[pl.BlockSpec((_BLK_N, _D), lambda i: (i, 0)),
                  full(1, _H), full(_D, _H), full(_H, _H), full(_H, _H)],
        out_specs=[pl.BlockSpec((_BLK_N, _H), lambda i: (i, 0))] * 3,
        out_shape=[jax.ShapeDtypeStruct((_N, _H), _f32)] * 3,
    )(x.astype(_f32), bias_eff, params["in_W"][:_D], lp0["e1_W"][:_H],
      lp0["e1_W"][_H:2 * _H])

    for li in range(3):
        lp = params["layers"][li]
        wr = lp["e1_W"][2 * _H:2 * _H + 1]
        we = lp["e1_W"][2 * _H + 1:]
        b1 = lp["e1_b"].reshape(1, _H)
        w2 = lp["e2_W"]
        b2 = lp["e2_b"].reshape(1, _H)
        wc4 = jnp.pad(lp["c_W"], ((0, 0), (0, 3)))
        bc4 = jnp.pad(lp["c_b"].reshape(1, 1), ((0, 0), (0, 3)))

        ga, gb, pd, ps = _sc_gather(A, B, pospad, dst3, src3)
        ma, mb, gd = _edge_call(ga, gb, pd, ps, ea, wr, we, b1, w2, b2, wc4, bc4)
        ms, ca = _sc_scatter(ma, mb, gd, dst3, z16, zp)

        w1h = lp["n1_W"][:_H]
        w1m0 = lp["n1_W"][_H:_H + _H // 2]
        w1m1 = lp["n1_W"][_H + _H // 2:]
        nb1 = lp["n1_b"].reshape(1, _H)
        n2w = lp["n2_W"]
        nb2 = lp["n2_b"].reshape(1, _H)

        if li < 2:
            lpn = params["layers"][li + 1]
            h, pospad, A, B = _node_call(
                h, ms[0], ms[1], ca, pospad,
                w1h, w1m0, w1m1, nb1, n2w, nb2,
                lpn["e1_W"][:_H], lpn["e1_W"][_H:2 * _H])
        else:
            ecw4 = jnp.pad(params["eps_c_W"], ((0, 0), (0, 1)))
            ecb4 = jnp.pad(params["eps_c_b"].reshape(1, 3), ((0, 0), (0, 1)))
            ec4, ef, pospad = _node_final_call(
                h, ms[0], ms[1], ca, pospad,
                w1h, w1m0, w1m1, nb1, n2w, nb2,
                ecw4, ecb4, params["eps_f_W"], params["eps_f_b"].reshape(1, _D))

    return ec4[:, :3], ef, pospad[:, :3]


# speedup vs baseline: 2.4692x; 2.4692x over previous
import functools

import jax
import jax.numpy as jnp
from jax import lax
from jax.experimental import pallas as pl
from jax.experimental.pallas import tpu as pltpu
from jax.experimental.pallas import tpu_sc as plsc

_N = 50000
_E = 800000
_D = 64
_H = 32
_NC = 2
_NS = 16
_NW = _NC * _NS
_CH = 1000
_CPW = _E // (_NW * _CH)
_SUB = 8
_SUBL = _CH // _SUB
_BLK_N = 2000
_BLK_E = 2000
_P = 8

_f32 = jnp.float32


def _silu(v):
    return v * (1.0 / (1.0 + jnp.exp(-v)))


def _embed_body(x_ref, be_ref, wx_ref, wd_ref, ws_ref, h_ref, a_ref, b_ref):
    h = jnp.dot(x_ref[...], wx_ref[...], preferred_element_type=_f32)
    h = h + be_ref[...]
    h_ref[...] = h
    a_ref[...] = jnp.dot(h, wd_ref[...], preferred_element_type=_f32)
    b_ref[...] = jnp.dot(h, ws_ref[...], preferred_element_type=_f32)


_RB = 400
_GRID_E = (_E // 16) // _RB

import numpy as _np

_MASK8 = _np.zeros((128, 16), _np.float32)
for _e in range(16):
    _MASK8[8 * _e:8 * _e + 8, _e] = 1.0
_SEL_A = _np.kron(_np.eye(16, dtype=_np.float32),
                  _np.vstack([_np.eye(16, dtype=_np.float32),
                              _np.zeros((16, 16), _np.float32)]))
_SEL_B = _np.kron(_np.eye(16, dtype=_np.float32),
                  _np.vstack([_np.zeros((16, 16), _np.float32),
                              _np.eye(16, dtype=_np.float32)]))
_ONE3T = _np.zeros((1, 128), _np.float32)
_ONE3T[0, 3::8] = 1.0


def _edge_body(ga_ref, gb_ref, pd_ref, ps_ref, ea_ref,
               m512_ref, wea_ref, bd16_ref, cg_ref,
               sa_ref, sb_ref, b1_ref, b2_ref, o3_ref, cbt_ref,
               ma_ref, mb_ref, gd_ref):
    diff = pd_ref[...] - ps_ref[...]
    sq = diff * diff
    r_x = jnp.dot(sq, m512_ref[...], preferred_element_type=_f32)
    ea_x = jnp.dot(ea_ref[...], wea_ref[...], preferred_element_type=_f32)
    x = ga_ref[...] + gb_ref[...] + r_x + ea_x + b1_ref[...]
    m = _silu(x)
    m2 = _silu(jnp.dot(m, bd16_ref[...], preferred_element_type=_f32)
               + b2_ref[...])
    g_b = jnp.dot(m2, cg_ref[...], preferred_element_type=_f32)
    gd_ref[...] = (g_b + cbt_ref[...]) * diff + o3_ref[...]
    ma_ref[...] = jnp.dot(m2, sa_ref[...], preferred_element_type=_f32)
    mb_ref[...] = jnp.dot(m2, sb_ref[...], preferred_element_type=_f32)


def _edge_call(ga, gb, pd, ps, ea, wr, we, b1, w2, b2, wc, bc):
    eye16 = jnp.eye(16, dtype=_f32)
    m512 = (jnp.asarray(_MASK8)[:, :, None] * wr.reshape(1, 1, _H)
            ).reshape(128, 512)
    wea = jnp.kron(eye16, we)
    bd16 = jnp.kron(eye16, w2)
    cg = jnp.kron(eye16, wc @ jnp.ones((1, 8), _f32))
    b1t = jnp.tile(b1, (1, 16))
    b2t = jnp.tile(b2, (1, 16))
    o3t = jnp.asarray(_ONE3T)
    cbt = jnp.tile(bc.reshape(1, 1), (1, 128))
    full = lambda r, c: pl.BlockSpec((r, c), lambda i: (0, 0))
    b512 = pl.BlockSpec((_RB, 512), lambda i: (i, 0))
    b256 = pl.BlockSpec((_RB, 256), lambda i: (i, 0))
    b128 = pl.BlockSpec((_RB, 128), lambda i: (i, 0))
    b64 = pl.BlockSpec((_RB, 64), lambda i: (i, 0))
    return pl.pallas_call(
        _edge_body,
        grid=(_GRID_E,),
        in_specs=[b512, b512, b128, b128, b64,
                  full(128, 512), full(64, 512), full(512, 512),
                  full(512, 128), full(512, 256), full(512, 256),
                  full(1, 512), full(1, 512), full(1, 128), full(1, 128)],
        out_specs=[b256, b256, b128],
        out_shape=[
            jax.ShapeDtypeStruct((_E // 16, 256), _f32),
            jax.ShapeDtypeStruct((_E // 16, 256), _f32),
            jax.ShapeDtypeStruct((_E // 16, 128), _f32),
        ],
    )(ga.reshape(_E // 16, 512), gb.reshape(_E // 16, 512),
      pd.reshape(_E // 16, 128), ps.reshape(_E // 16, 128),
      ea.reshape(_E // 16, 64),
      m512, wea, bd16, cg,
      jnp.asarray(_SEL_A), jnp.asarray(_SEL_B), b1t, b2t, o3t, cbt)


def _node_body(h_ref, ms0_ref, ms1_ref, ca_ref, pp_ref,
               w1h_ref, w1m0_ref, w1m1_ref, b1_ref, w2_ref, b2_ref,
               wdn_ref, wsn_ref,
               h_out, pp_out, a_out, b_out):
    cacc = ca_ref[...]
    deg = jnp.maximum(cacc[:, 3:4], 1.0)
    inv = 1.0 / deg
    hn = _silu(jnp.dot(h_ref[...], w1h_ref[...], preferred_element_type=_f32)
               + jnp.dot(ms0_ref[...] * inv, w1m0_ref[...], preferred_element_type=_f32)
               + jnp.dot(ms1_ref[...] * inv, w1m1_ref[...], preferred_element_type=_f32)
               + b1_ref[...])
    hnew = jnp.dot(hn, w2_ref[...], preferred_element_type=_f32) + b2_ref[...]
    h_out[...] = hnew
    col = lax.broadcasted_iota(jnp.int32, (1, _P), 1)
    mask3 = (col < 3).astype(_f32)
    pp_out[...] = pp_ref[...] + (cacc * mask3) * inv
    a_out[...] = jnp.dot(hnew, wdn_ref[...], preferred_element_type=_f32)
    b_out[...] = jnp.dot(hnew, wsn_ref[...], preferred_element_type=_f32)


def _node_call(h, ms0, ms1, ca, pp, w1h, w1m0, w1m1, b1, w2, b2, wdn, wsn):
    grid = _N // _BLK_N
    full = lambda r, c: pl.BlockSpec((r, c), lambda i: (0, 0))
    row32 = pl.BlockSpec((_BLK_N, _H), lambda i: (i, 0))
    row16 = pl.BlockSpec((_BLK_N, _H // 2), lambda i: (i, 0))
    rowp = pl.BlockSpec((_BLK_N, _P), lambda i: (i, 0))
    return pl.pallas_call(
        _node_body,
        grid=(grid,),
        in_specs=[row32, row16, row16, rowp, rowp,
                  full(_H, _H), full(_H // 2, _H), full(_H // 2, _H),
                  full(1, _H),
                  full(_H, _H), full(1, _H), full(_H, _H), full(_H, _H)],
        out_specs=[row32, rowp, row32, row32],
        out_shape=[
            jax.ShapeDtypeStruct((_N, _H), _f32),
            jax.ShapeDtypeStruct((_N, _P), _f32),
            jax.ShapeDtypeStruct((_N, _H), _f32),
            jax.ShapeDtypeStruct((_N, _H), _f32),
        ],
    )(h, ms0, ms1, ca, pp, w1h, w1m0, w1m1, b1, w2, b2, wdn, wsn)


def _node_final_body(h_ref, ms0_ref, ms1_ref, ca_ref, pp_ref,
                     w1h_ref, w1m0_ref, w1m1_ref, b1_ref, w2_ref, b2_ref,
                     ecw_ref, ecb_ref, efw_ref, efb_ref,
                     ec_out, ef_out, pp_out):
    cacc = ca_ref[...]
    deg = jnp.maximum(cacc[:, 3:4], 1.0)
    inv = 1.0 / deg
    hn = _silu(jnp.dot(h_ref[...], w1h_ref[...], preferred_element_type=_f32)
               + jnp.dot(ms0_ref[...] * inv, w1m0_ref[...], preferred_element_type=_f32)
               + jnp.dot(ms1_ref[...] * inv, w1m1_ref[...], preferred_element_type=_f32)
               + b1_ref[...])
    hnew = jnp.dot(hn, w2_ref[...], preferred_element_type=_f32) + b2_ref[...]
    col = lax.broadcasted_iota(jnp.int32, (1, _P), 1)
    mask3 = (col < 3).astype(_f32)
    pp_out[...] = pp_ref[...] + (cacc * mask3) * inv
    ec_out[...] = jnp.dot(hnew, ecw_ref[...], preferred_element_type=_f32) + ecb_ref[...]
    ef_out[...] = jnp.dot(hnew, efw_ref[...], preferred_element_type=_f32) + efb_ref[...]


def _node_final_call(h, ms0, ms1, ca, pp, w1h, w1m0, w1m1, b1, w2, b2,
                     ecw4, ecb4, efw, efb):
    grid = _N // _BLK_N
    full = lambda r, c: pl.BlockSpec((r, c), lambda i: (0, 0))
    row32 = pl.BlockSpec((_BLK_N, _H), lambda i: (i, 0))
    row16 = pl.BlockSpec((_BLK_N, _H // 2), lambda i: (i, 0))
    rowp = pl.BlockSpec((_BLK_N, _P), lambda i: (i, 0))
    row4 = pl.BlockSpec((_BLK_N, 4), lambda i: (i, 0))
    row64 = pl.BlockSpec((_BLK_N, _D), lambda i: (i, 0))
    return pl.pallas_call(
        _node_final_body,
        grid=(grid,),
        in_specs=[row32, row16, row16, rowp, rowp,
                  full(_H, _H), full(_H // 2, _H), full(_H // 2, _H),
                  full(1, _H),
                  full(_H, _H), full(1, _H),
                  full(_H, 4), full(1, 4), full(_H, _D), full(1, _D)],
        out_specs=[row4, row64, rowp],
        out_shape=[
            jax.ShapeDtypeStruct((_N, 4), _f32),
            jax.ShapeDtypeStruct((_N, _D), _f32),
            jax.ShapeDtypeStruct((_N, _P), _f32),
        ],
    )(h, ms0, ms1, ca, pp, w1h, w1m0, w1m1, b1, w2, b2, ecw4, ecb4, efw, efb)


def _sc_gather_body(a_hbm, b_hbm, pp_hbm, dst3, src3,
               ga_out, gb_out, pd_out, ps_out,
               idxd, idxs, bufa, bufb, bufpd, bufps, sem):
    c = lax.axis_index("c")
    s = lax.axis_index("s")
    wid = s * _NC + c

    def chunk(j, carry):
        cg = wid * _CPW + j
        base = cg * _CH
        pltpu.sync_copy(dst3.at[cg], idxd)
        pltpu.sync_copy(src3.at[cg], idxs)

        descs = []
        for jj in range(_SUB):
            rows = pl.ds(jj * _SUBL, _SUBL)
            descs.append(pltpu.async_copy(a_hbm.at[idxd.at[jj]], bufa.at[rows], sem))
            descs.append(pltpu.async_copy(b_hbm.at[idxs.at[jj]], bufb.at[rows], sem))
            descs.append(pltpu.async_copy(pp_hbm.at[idxd.at[jj]], bufpd.at[rows], sem))
            descs.append(pltpu.async_copy(pp_hbm.at[idxs.at[jj]], bufps.at[rows], sem))
        for d in descs:
            d.wait()
        pltpu.sync_copy(bufa, ga_out.at[pl.ds(base, _CH)])
        pltpu.sync_copy(bufb, gb_out.at[pl.ds(base, _CH)])
        pltpu.sync_copy(bufpd, pd_out.at[pl.ds(base, _CH)])
        pltpu.sync_copy(bufps, ps_out.at[pl.ds(base, _CH)])
        return carry

    lax.fori_loop(0, _CPW, chunk, 0)


_NCHUNK = _E // _CH
_CPS = _NCHUNK // _NS


def _sc_scatter_body(ma_hbm, mb_hbm, gd_hbm, dst3, z16, zp, ms_out, ca_out,
                idxd, bufm, bufg, msacc, caacc, sem):
    c = lax.axis_index("c")
    s = lax.axis_index("s")

    @pl.when(s == 0)
    def _init():
        pltpu.sync_copy(z16, msacc)

    @pl.when((s == 1) & (c == 1))
    def _init2():
        pltpu.sync_copy(zp, caacc)

    plsc.subcore_barrier()

    def chunk(j, carry):
        cg = s * _CPS + j
        rows = pl.ds(cg * _CH, _CH)
        pltpu.sync_copy(dst3.at[cg], idxd)

        @pl.when(c == 0)
        def _stage_a():
            pltpu.sync_copy(ma_hbm.at[rows], bufm)

        @pl.when(c == 1)
        def _stage_b():
            pltpu.sync_copy(mb_hbm.at[rows], bufm)
            pltpu.sync_copy(gd_hbm.at[rows], bufg)

        for jj in range(_SUB):
            sub = pl.ds(jj * _SUBL, _SUBL)
            pltpu.sync_copy(bufm.at[sub], msacc.at[idxd.at[jj]], add=True)

        @pl.when(c == 1)
        def _scat_g():
            for jj in range(_SUB):
                sub = pl.ds(jj * _SUBL, _SUBL)
                pltpu.sync_copy(bufg.at[sub], caacc.at[idxd.at[jj]], add=True)

        return carry

    lax.fori_loop(0, _CPS, chunk, 0)
    plsc.subcore_barrier()

    @pl.when(s == 0)
    def _writeout():
        pltpu.sync_copy(msacc, ms_out.at[c])

    @pl.when((s == 1) & (c == 1))
    def _writeout2():
        pltpu.sync_copy(caacc, ca_out)


_SC_KERNELS = {}


def _get_sc_kernels():
    if not _SC_KERNELS:
        mesh = plsc.VectorSubcoreMesh(core_axis_name="c", subcore_axis_name="s",
                                      num_cores=_NC, num_subcores=_NS)
        cp = pltpu.CompilerParams(use_tc_tiling_on_sc=False)
        _SC_KERNELS["gather"] = pl.kernel(
            _sc_gather_body,
            out_type=[
                jax.ShapeDtypeStruct((_E, _H), _f32),
                jax.ShapeDtypeStruct((_E, _H), _f32),
                jax.ShapeDtypeStruct((_E, _P), _f32),
                jax.ShapeDtypeStruct((_E, _P), _f32),
            ],
            mesh=mesh,
            scratch_types=[
                pltpu.VMEM((_SUB, _SUBL), jnp.int32),
                pltpu.VMEM((_SUB, _SUBL), jnp.int32),
                pltpu.VMEM((_CH, _H), _f32),
                pltpu.VMEM((_CH, _H), _f32),
                pltpu.VMEM((_CH, _P), _f32),
                pltpu.VMEM((_CH, _P), _f32),
                pltpu.SemaphoreType.DMA,
            ],
            compiler_params=cp,
        )
        _SC_KERNELS["scatter"] = pl.kernel(
            _sc_scatter_body,
            out_type=[
                jax.ShapeDtypeStruct((_NC, _N, _H // 2), _f32),
                jax.ShapeDtypeStruct((_N, _P), _f32),
            ],
            mesh=mesh,
            scratch_types=[
                pltpu.VMEM((_SUB, _SUBL), jnp.int32),
                pltpu.VMEM((_CH, _H // 2), _f32),
                pltpu.VMEM((_CH, _P), _f32),
                pltpu.VMEM_SHARED((_N, _H // 2), _f32),
                pltpu.VMEM_SHARED((_N, _P), _f32),
                pltpu.SemaphoreType.DMA,
            ],
            compiler_params=cp,
        )
    return _SC_KERNELS["gather"], _SC_KERNELS["scatter"]


def kernel(x, pos, edge_index, edge_attr, t, params):
    src = edge_index[0].astype(jnp.int32)
    dst = edge_index[1].astype(jnp.int32)
    dst3 = dst.reshape(_NW * _CPW, _SUB, _SUBL)
    src3 = src.reshape(_NW * _CPW, _SUB, _SUBL)
    pospad = jnp.pad(pos.astype(_f32), ((0, 0), (0, _P - 3)))
    ea = edge_attr.astype(_f32)

    z16 = jnp.zeros((_N, _H // 2), _f32)
    zp = jnp.zeros((_N, _P), _f32)

    lp0 = params["layers"][0]
    bias_eff = (t[0] * params["in_W"][_D] + params["in_b"]).reshape(1, _H)

    grid = _N // _BLK_N
    full = lambda r, c: pl.BlockSpec((r, c), lambda i: (0, 0))
    h, A, B = pl.pallas_call(
        _embed_body,
        grid=(grid,),
        in_specs=[pl.BlockSpec((_BLK_N, _D), lambda i: (i, 0)),
                  full(1, _H), full(_D, _H), full(_H, _H), full(_H, _H)],
        out_specs=[pl.BlockSpec((_BLK_N, _H), lambda i: (i, 0))] * 3,
        out_shape=[jax.ShapeDtypeStruct((_N, _H), _f32)] * 3,
    )(x.astype(_f32), bias_eff, params["in_W"][:_D], lp0["e1_W"][:_H],
      lp0["e1_W"][_H:2 * _H])

    for li in range(3):
        lp = params["layers"][li]
        wr = lp["e1_W"][2 * _H:2 * _H + 1]
        we = lp["e1_W"][2 * _H + 1:]
        b1 = lp["e1_b"].reshape(1, _H)
        w2 = lp["e2_W"]
        b2 = lp["e2_b"].reshape(1, _H)

        sc_gather, sc_scatter = _get_sc_kernels()
        ga, gb, pd, ps = sc_gather(A, B, pospad, dst3, src3)
        ma, mb, gd = _edge_call(ga, gb, pd, ps, ea, wr, we, b1, w2, b2,
                                lp["c_W"], lp["c_b"])
        ms, ca = sc_scatter(ma.reshape(_E, _H // 2), mb.reshape(_E, _H // 2),
                            gd.reshape(_E, _P), dst3, z16, zp)

        w1h = lp["n1_W"][:_H]
        w1m0 = lp["n1_W"][_H:_H + _H // 2]
        w1m1 = lp["n1_W"][_H + _H // 2:]
        nb1 = lp["n1_b"].reshape(1, _H)
        n2w = lp["n2_W"]
        nb2 = lp["n2_b"].reshape(1, _H)

        if li < 2:
            lpn = params["layers"][li + 1]
            h, pospad, A, B = _node_call(
                h, ms[0], ms[1], ca, pospad,
                w1h, w1m0, w1m1, nb1, n2w, nb2,
                lpn["e1_W"][:_H], lpn["e1_W"][_H:2 * _H])
        else:
            ecw4 = jnp.pad(params["eps_c_W"], ((0, 0), (0, 1)))
            ecb4 = jnp.pad(params["eps_c_b"].reshape(1, 3), ((0, 0), (0, 1)))
            ec4, ef, pospad = _node_final_call(
                h, ms[0], ms[1], ca, pospad,
                w1h, w1m0, w1m1, nb1, n2w, nb2,
                ecw4, ecb4, params["eps_f_W"], params["eps_f_b"].reshape(1, _D))

    return ec4[:, :3], ef, pospad[:, :3]
